# baseline (speedup 1.0000x reference)
import functools

import jax
import jax.numpy as jnp
from jax import lax
from jax.experimental import pallas as pl
from jax.experimental.pallas import tpu as pltpu
from jax.experimental.pallas import tpu_sc as plsc

E = 64
TOP_K = 2
H = 1024
I = 2048
TM = 256
NW = 32
TPW = 64
CH = 32


def _gmm_body(tile_ref, eid_ref, st_ref, en_ref,
              xs_ref, w1_ref, b1_ref, w2_ref, b2_ref, ys_ref):
    s = pl.program_id(0)
    tile = tile_ref[s]
    st = st_ref[s]
    en = en_ref[s]

    @pl.when(st < en)
    def _():
        rows = tile * TM + lax.broadcasted_iota(jnp.int32, (TM, 1), 0)
        mask = (rows >= st) & (rows < en)
        xb = xs_ref[...].astype(jnp.bfloat16)
        w1 = w1_ref[0].astype(jnp.bfloat16)
        h = lax.dot_general(xb, w1, (((1,), (1,)), ((), ())),
                            preferred_element_type=jnp.float32)
        h = h + b1_ref[0]
        h = 0.5 * h * (1.0 + lax.erf(h * 0.7071067811865476))
        y = lax.dot_general(h.astype(jnp.bfloat16), w2_ref[0].astype(jnp.bfloat16),
                            (((1,), (1,)), ((), ())),
                            preferred_element_type=jnp.float32)
        y = y + b2_ref[0]
        ys_ref[...] = jnp.where(mask, y, ys_ref[...])


def _grouped_mlp(xs, c_fc_w, c_fc_b, c_proj_w, c_proj_b,
                 step_tile, step_eid, step_st, step_en, grid_steps):
    n = xs.shape[0]
    grid_spec = pltpu.PrefetchScalarGridSpec(
        num_scalar_prefetch=4,
        grid=(grid_steps,),
        in_specs=[
            pl.BlockSpec((TM, H), lambda s, t, e, a, b: (t[s], 0)),
            pl.BlockSpec((1, I, H), lambda s, t, e, a, b: (e[s], 0, 0)),
            pl.BlockSpec((1, 1, I), lambda s, t, e, a, b: (e[s], 0, 0)),
            pl.BlockSpec((1, H, I), lambda s, t, e, a, b: (e[s], 0, 0)),
            pl.BlockSpec((1, 1, H), lambda s, t, e, a, b: (e[s], 0, 0)),
        ],
        out_specs=pl.BlockSpec((TM, H), lambda s, t, e, a, b: (t[s], 0)),
    )
    return pl.pallas_call(
        _gmm_body,
        grid_spec=grid_spec,
        out_shape=jax.ShapeDtypeStruct((n, H), jnp.float32),
        compiler_params=pltpu.CompilerParams(
            dimension_semantics=("arbitrary",),
        ),
    )(step_tile, step_eid, step_st, step_en,
      xs, c_fc_w, c_fc_b.reshape(E, 1, I), c_proj_w, c_proj_b.reshape(E, 1, H))


def _sc_worker_id():
    return lax.axis_index("s") * 2 + lax.axis_index("c")


def _sc_dispatch_body(xf_hbm, p0_hbm, p1_hbm, xs_hbm, idx_v, row_v, sem):
    base = _sc_worker_id() * TPW
    pltpu.sync_copy(p0_hbm.at[pl.ds(base, CH)], idx_v.at[0])
    pltpu.sync_copy(p0_hbm.at[pl.ds(base + CH, CH)], idx_v.at[1])
    pltpu.sync_copy(p1_hbm.at[pl.ds(base, CH)], idx_v.at[2])
    pltpu.sync_copy(p1_hbm.at[pl.ds(base + CH, CH)], idx_v.at[3])
    for c in range(TPW // CH):
        pltpu.sync_copy(xf_hbm.at[pl.ds(base + c * CH, CH)], row_v)
        a = pltpu.async_copy(row_v, xs_hbm.at[idx_v.at[c]], sem)
        b = pltpu.async_copy(row_v, xs_hbm.at[idx_v.at[2 + c]], sem)
        a.wait()
        b.wait()


@functools.lru_cache(maxsize=None)
def _sc_kernels(t_tokens):
    mesh = plsc.VectorSubcoreMesh(core_axis_name="c", subcore_axis_name="s")
    dispatch = pl.kernel(
        _sc_dispatch_body,
        mesh=mesh,
        out_type=jax.ShapeDtypeStruct((TOP_K * t_tokens, H), jnp.float32),
        scratch_types=[
            pltpu.VMEM((4, CH), jnp.int32),
            pltpu.VMEM((CH, H), jnp.float32),
            pltpu.SemaphoreType.DMA,
        ],
    )
    combine = pl.kernel(
        _sc_combine_body,
        mesh=mesh,
        out_type=jax.ShapeDtypeStruct((t_tokens, H), jnp.float32),
        scratch_types=[
            pltpu.VMEM((4, CH), jnp.int32),
            pltpu.VMEM((CH, 16), jnp.float32),
            pltpu.VMEM((CH, 16), jnp.float32),
            pltpu.VMEM((CH, H), jnp.float32),
            pltpu.VMEM((CH, H), jnp.float32),
            pltpu.SemaphoreType.DMA,
        ],
    )
    return dispatch, combine


def _sc_combine_body(ys_hbm, p0_hbm, p1_hbm, grw0_hbm, grw1_hbm, out_hbm,
                     idx_v, g0_v, g1_v, buf0, buf1, sem):
    base = _sc_worker_id() * TPW
    pltpu.sync_copy(p0_hbm.at[pl.ds(base, CH)], idx_v.at[0])
    pltpu.sync_copy(p0_hbm.at[pl.ds(base + CH, CH)], idx_v.at[1])
    pltpu.sync_copy(p1_hbm.at[pl.ds(base, CH)], idx_v.at[2])
    pltpu.sync_copy(p1_hbm.at[pl.ds(base + CH, CH)], idx_v.at[3])
    for c in range(TPW // CH):
        a = pltpu.async_copy(ys_hbm.at[idx_v.at[c]], buf0, sem)
        b = pltpu.async_copy(ys_hbm.at[idx_v.at[2 + c]], buf1, sem)
        pltpu.sync_copy(grw0_hbm.at[pl.ds(base + c * CH, CH)], g0_v)
        pltpu.sync_copy(grw1_hbm.at[pl.ds(base + c * CH, CH)], g1_v)
        a.wait()
        b.wait()

        def tok_body(j, carry):
            g0 = g0_v[j, :]
            g1 = g1_v[j, :]
            for k in range(H // 16):
                sl = pl.ds(k * 16, 16)
                buf0[j, sl] = buf0[j, sl] * g0 + buf1[j, sl] * g1
            return carry

        lax.fori_loop(0, CH, tok_body, 0)
        pltpu.sync_copy(buf0, out_hbm.at[pl.ds(base + c * CH, CH)])


def kernel(x, gate_w, c_fc_w, c_fc_b, c_proj_w, c_proj_b):
    orig_shape = x.shape
    xf = x.reshape(-1, H)
    t_tokens = xf.shape[0]
    n = t_tokens * TOP_K
    nt = n // TM

    router_logits = xf @ gate_w.T
    vals, sel = lax.top_k(router_logits, TOP_K)
    rw = jax.nn.softmax(vals.astype(jnp.float32), axis=-1)
    sel0, sel1 = sel[:, 0], sel[:, 1]
    rw0, rw1 = rw[:, 0], rw[:, 1]

    eids = jnp.arange(E, dtype=jnp.int32)
    oh0 = (sel0[:, None] == eids[None, :]).astype(jnp.int32)
    oh1 = (sel1[:, None] == eids[None, :]).astype(jnp.int32)
    c0 = jnp.cumsum(oh0, axis=0)
    c1 = jnp.cumsum(oh1, axis=0)
    total0 = c0[-1]
    counts = total0 + c1[-1]
    ends = jnp.cumsum(counts)
    starts = ends - counts
    rank0 = jnp.take_along_axis(c0, sel0[:, None], axis=1)[:, 0]
    rank1 = jnp.take_along_axis(c1, sel1[:, None], axis=1)[:, 0]
    pos0 = (starts[sel0] + rank0 - 1).astype(jnp.int32)
    pos1 = (starts[sel1] + total0[sel1] + rank1 - 1).astype(jnp.int32)

    t0 = starts // TM
    t1 = jnp.maximum(ends - 1, 0) // TM
    u = jnp.where(counts > 0, t1 - t0 + 1, 0)
    cum_u = jnp.cumsum(u)
    grid_steps = nt + E - 1
    s_idx = jnp.minimum(jnp.arange(grid_steps), cum_u[-1] - 1)
    eid = jnp.searchsorted(cum_u, s_idx, side="right").astype(jnp.int32)
    u_excl = cum_u - u
    step_tile = (t0[eid] + (s_idx - u_excl[eid])).astype(jnp.int32)
    is_pad = jnp.arange(grid_steps) >= cum_u[-1]
    step_st = jnp.where(is_pad, 0, starts[eid]).astype(jnp.int32)
    step_en = jnp.where(is_pad, 0, ends[eid]).astype(jnp.int32)

    sc_dispatch, sc_combine = _sc_kernels(t_tokens)
    xs = sc_dispatch(xf, pos0, pos1)
    ys = _grouped_mlp(xs, c_fc_w, c_fc_b, c_proj_w, c_proj_b,
                      step_tile, eid, step_st, step_en, grid_steps)
    grw0 = jnp.broadcast_to(rw0[:, None], (t_tokens, 16))
    grw1 = jnp.broadcast_to(rw1[:, None], (t_tokens, 16))
    out = sc_combine(ys, pos0, pos1, grw0, grw1)

    return (out.reshape(orig_shape), router_logits)


# speedup vs baseline: 1.3327x; 1.3327x over previous
import functools

import jax
import jax.numpy as jnp
from jax import lax
from jax.experimental import pallas as pl
from jax.experimental.pallas import tpu as pltpu
from jax.experimental.pallas import tpu_sc as plsc

E = 64
TOP_K = 2
H = 1024
I = 2048
TM = 256
NW = 32
TPW = 64
CH = 32


def _gmm_body(tile_ref, eid_ref, st_ref, en_ref,
              xs_ref, w1_ref, b1_ref, w2_ref, b2_ref, ys_ref):
    s = pl.program_id(0)
    tile = tile_ref[s]
    st = st_ref[s]
    en = en_ref[s]

    @pl.when(st < en)
    def _():
        rows = tile * TM + lax.broadcasted_iota(jnp.int32, (TM, 1), 0)
        mask = (rows >= st) & (rows < en)
        xb = xs_ref[...].astype(jnp.bfloat16)
        w1 = w1_ref[0].astype(jnp.bfloat16)
        h = lax.dot_general(xb, w1, (((1,), (1,)), ((), ())),
                            preferred_element_type=jnp.float32)
        h = h + b1_ref[0]
        h = 0.5 * h * (1.0 + lax.erf(h * 0.7071067811865476))
        y = lax.dot_general(h.astype(jnp.bfloat16), w2_ref[0].astype(jnp.bfloat16),
                            (((1,), (1,)), ((), ())),
                            preferred_element_type=jnp.float32)
        y = y + b2_ref[0]
        ys_ref[...] = jnp.where(mask, y, ys_ref[...])


def _grouped_mlp(xs, c_fc_w, c_fc_b, c_proj_w, c_proj_b,
                 step_tile, step_eid, step_st, step_en, grid_steps):
    n = xs.shape[0]
    grid_spec = pltpu.PrefetchScalarGridSpec(
        num_scalar_prefetch=4,
        grid=(grid_steps,),
        in_specs=[
            pl.BlockSpec((TM, H), lambda s, t, e, a, b: (t[s], 0)),
            pl.BlockSpec((1, I, H), lambda s, t, e, a, b: (e[s], 0, 0)),
            pl.BlockSpec((1, 1, I), lambda s, t, e, a, b: (e[s], 0, 0)),
            pl.BlockSpec((1, H, I), lambda s, t, e, a, b: (e[s], 0, 0)),
            pl.BlockSpec((1, 1, H), lambda s, t, e, a, b: (e[s], 0, 0)),
        ],
        out_specs=pl.BlockSpec((TM, H), lambda s, t, e, a, b: (t[s], 0)),
    )
    return pl.pallas_call(
        _gmm_body,
        grid_spec=grid_spec,
        out_shape=jax.ShapeDtypeStruct((n, H), jnp.float32),
        compiler_params=pltpu.CompilerParams(
            dimension_semantics=("arbitrary",),
        ),
    )(step_tile, step_eid, step_st, step_en,
      xs, c_fc_w, c_fc_b.reshape(E, 1, I), c_proj_w, c_proj_b.reshape(E, 1, H))


def _sc_worker_id():
    return lax.axis_index("s") * 2 + lax.axis_index("c")


def _sc_dispatch_body(xf_hbm, p0_hbm, p1_hbm, xs_hbm, idx_v, row_v, sem):
    base = _sc_worker_id() * TPW
    pltpu.sync_copy(p0_hbm.at[pl.ds(base, CH)], idx_v.at[0])
    pltpu.sync_copy(p0_hbm.at[pl.ds(base + CH, CH)], idx_v.at[1])
    pltpu.sync_copy(p1_hbm.at[pl.ds(base, CH)], idx_v.at[2])
    pltpu.sync_copy(p1_hbm.at[pl.ds(base + CH, CH)], idx_v.at[3])
    for c in range(TPW // CH):
        pltpu.sync_copy(xf_hbm.at[pl.ds(base + c * CH, CH)], row_v)
        a = pltpu.async_copy(row_v, xs_hbm.at[idx_v.at[c]], sem)
        b = pltpu.async_copy(row_v, xs_hbm.at[idx_v.at[2 + c]], sem)
        a.wait()
        b.wait()


@functools.lru_cache(maxsize=None)
def _sc_kernels(t_tokens):
    mesh = plsc.VectorSubcoreMesh(core_axis_name="c", subcore_axis_name="s")
    dispatch = pl.kernel(
        _sc_dispatch_body,
        mesh=mesh,
        out_type=jax.ShapeDtypeStruct((TOP_K * t_tokens, H), jnp.float32),
        scratch_types=[
            pltpu.VMEM((4, CH), jnp.int32),
            pltpu.VMEM((CH, H), jnp.float32),
            pltpu.SemaphoreType.DMA,
        ],
    )
    combine = pl.kernel(
        _sc_combine_body,
        mesh=mesh,
        out_type=jax.ShapeDtypeStruct((t_tokens, H), jnp.float32),
        scratch_types=[
            pltpu.VMEM((4, CH), jnp.int32),
            pltpu.VMEM((CH, 16), jnp.float32),
            pltpu.VMEM((CH, 16), jnp.float32),
            pltpu.VMEM((CH, H), jnp.float32),
            pltpu.VMEM((CH, H), jnp.float32),
            pltpu.SemaphoreType.DMA,
        ],
    )
    return dispatch, combine


def _sc_combine_body(ys_hbm, p0_hbm, p1_hbm, grw0_hbm, grw1_hbm, out_hbm,
                     idx_v, g0_v, g1_v, buf0, buf1, sem):
    base = _sc_worker_id() * TPW
    pltpu.sync_copy(p0_hbm.at[pl.ds(base, CH)], idx_v.at[0])
    pltpu.sync_copy(p0_hbm.at[pl.ds(base + CH, CH)], idx_v.at[1])
    pltpu.sync_copy(p1_hbm.at[pl.ds(base, CH)], idx_v.at[2])
    pltpu.sync_copy(p1_hbm.at[pl.ds(base + CH, CH)], idx_v.at[3])
    for c in range(TPW // CH):
        a = pltpu.async_copy(ys_hbm.at[idx_v.at[c]], buf0, sem)
        b = pltpu.async_copy(ys_hbm.at[idx_v.at[2 + c]], buf1, sem)
        pltpu.sync_copy(grw0_hbm.at[pl.ds(base + c * CH, CH)], g0_v)
        pltpu.sync_copy(grw1_hbm.at[pl.ds(base + c * CH, CH)], g1_v)
        a.wait()
        b.wait()

        def tok_body(j, carry):
            g0 = g0_v[j, :]
            g1 = g1_v[j, :]
            for k in range(H // 16):
                sl = pl.ds(k * 16, 16)
                buf0[j, sl] = buf0[j, sl] * g0 + buf1[j, sl] * g1
            return carry

        lax.fori_loop(0, CH, tok_body, 0)
        pltpu.sync_copy(buf0, out_hbm.at[pl.ds(base + c * CH, CH)])


def kernel(x, gate_w, c_fc_w, c_fc_b, c_proj_w, c_proj_b):
    orig_shape = x.shape
    xf = x.reshape(-1, H)
    t_tokens = xf.shape[0]
    n = t_tokens * TOP_K
    nt = n // TM

    router_logits = xf @ gate_w.T
    eids = jnp.arange(E, dtype=jnp.int32)
    v0 = jnp.max(router_logits, axis=-1)
    sel0 = jnp.argmax(router_logits, axis=-1).astype(jnp.int32)
    masked = jnp.where(eids[None, :] == sel0[:, None], -jnp.inf, router_logits)
    v1 = jnp.max(masked, axis=-1)
    sel1 = jnp.argmax(masked, axis=-1).astype(jnp.int32)
    rw0 = 1.0 / (1.0 + jnp.exp(v1 - v0))
    rw1 = 1.0 - rw0

    flat_sel = jnp.stack([sel0, sel1], axis=1).reshape(-1)
    sorted_idx = jnp.argsort(flat_sel)
    inv = jnp.zeros((n,), jnp.int32).at[sorted_idx].set(
        jnp.arange(n, dtype=jnp.int32))
    pos01 = inv.reshape(t_tokens, TOP_K)
    pos0, pos1 = pos01[:, 0], pos01[:, 1]
    counts = jnp.sum(flat_sel[:, None] == eids[None, :], axis=0)
    ends = jnp.cumsum(counts)
    starts = ends - counts

    t0 = starts // TM
    t1 = jnp.maximum(ends - 1, 0) // TM
    u = jnp.where(counts > 0, t1 - t0 + 1, 0)
    cum_u = jnp.cumsum(u)
    grid_steps = nt + E - 1
    s_idx = jnp.minimum(jnp.arange(grid_steps), cum_u[-1] - 1)
    eid = jnp.searchsorted(cum_u, s_idx, side="right").astype(jnp.int32)
    u_excl = cum_u - u
    step_tile = (t0[eid] + (s_idx - u_excl[eid])).astype(jnp.int32)
    is_pad = jnp.arange(grid_steps) >= cum_u[-1]
    step_st = jnp.where(is_pad, 0, starts[eid]).astype(jnp.int32)
    step_en = jnp.where(is_pad, 0, ends[eid]).astype(jnp.int32)

    sc_dispatch, sc_combine = _sc_kernels(t_tokens)
    xs = sc_dispatch(xf, pos0, pos1)
    ys = _grouped_mlp(xs, c_fc_w, c_fc_b, c_proj_w, c_proj_b,
                      step_tile, eid, step_st, step_en, grid_steps)
    grw0 = jnp.broadcast_to(rw0[:, None], (t_tokens, 16))
    grw1 = jnp.broadcast_to(rw1[:, None], (t_tokens, 16))
    out = sc_combine(ys, pos0, pos1, grw0, grw1)

    return (out.reshape(orig_shape), router_logits)


# speedup vs baseline: 1.3349x; 1.0017x over previous
import functools

import jax
import jax.numpy as jnp
from jax import lax
from jax.experimental import pallas as pl
from jax.experimental.pallas import tpu as pltpu
from jax.experimental.pallas import tpu_sc as plsc

E = 64
TOP_K = 2
H = 1024
I = 2048
TM = 512
NW = 32
TPW = 64
CH = 32


def _gmm_body(tile_ref, eid_ref, st_ref, en_ref,
              xs_ref, w1_ref, b1_ref, w2_ref, b2_ref, ys_ref):
    s = pl.program_id(0)
    tile = tile_ref[s]
    st = st_ref[s]
    en = en_ref[s]

    @pl.when(st < en)
    def _():
        rows = tile * TM + lax.broadcasted_iota(jnp.int32, (TM, 1), 0)
        mask = (rows >= st) & (rows < en)
        xb = xs_ref[...].astype(jnp.bfloat16)
        w1 = w1_ref[0].astype(jnp.bfloat16)
        h = lax.dot_general(xb, w1, (((1,), (1,)), ((), ())),
                            preferred_element_type=jnp.float32)
        h = h + b1_ref[0]
        h = 0.5 * h * (1.0 + lax.erf(h * 0.7071067811865476))
        y = lax.dot_general(h.astype(jnp.bfloat16), w2_ref[0].astype(jnp.bfloat16),
                            (((1,), (1,)), ((), ())),
                            preferred_element_type=jnp.float32)
        y = y + b2_ref[0]
        ys_ref[...] = jnp.where(mask, y, ys_ref[...])


def _grouped_mlp(xs, c_fc_w, c_fc_b, c_proj_w, c_proj_b,
                 step_tile, step_eid, step_st, step_en, grid_steps):
    n = xs.shape[0]
    grid_spec = pltpu.PrefetchScalarGridSpec(
        num_scalar_prefetch=4,
        grid=(grid_steps,),
        in_specs=[
            pl.BlockSpec((TM, H), lambda s, t, e, a, b: (t[s], 0)),
            pl.BlockSpec((1, I, H), lambda s, t, e, a, b: (e[s], 0, 0)),
            pl.BlockSpec((1, 1, I), lambda s, t, e, a, b: (e[s], 0, 0)),
            pl.BlockSpec((1, H, I), lambda s, t, e, a, b: (e[s], 0, 0)),
            pl.BlockSpec((1, 1, H), lambda s, t, e, a, b: (e[s], 0, 0)),
        ],
        out_specs=pl.BlockSpec((TM, H), lambda s, t, e, a, b: (t[s], 0)),
    )
    return pl.pallas_call(
        _gmm_body,
        grid_spec=grid_spec,
        out_shape=jax.ShapeDtypeStruct((n, H), jnp.float32),
        compiler_params=pltpu.CompilerParams(
            dimension_semantics=("arbitrary",),
        ),
    )(step_tile, step_eid, step_st, step_en,
      xs, c_fc_w, c_fc_b.reshape(E, 1, I), c_proj_w, c_proj_b.reshape(E, 1, H))


def _sc_worker_id():
    return lax.axis_index("s") * 2 + lax.axis_index("c")


def _sc_dispatch_body(xf_hbm, p0_hbm, p1_hbm, xs_hbm, idx_v, row_v, sem):
    base = _sc_worker_id() * TPW
    pltpu.sync_copy(p0_hbm.at[pl.ds(base, CH)], idx_v.at[0])
    pltpu.sync_copy(p0_hbm.at[pl.ds(base + CH, CH)], idx_v.at[1])
    pltpu.sync_copy(p1_hbm.at[pl.ds(base, CH)], idx_v.at[2])
    pltpu.sync_copy(p1_hbm.at[pl.ds(base + CH, CH)], idx_v.at[3])
    for c in range(TPW // CH):
        pltpu.sync_copy(xf_hbm.at[pl.ds(base + c * CH, CH)], row_v)
        a = pltpu.async_copy(row_v, xs_hbm.at[idx_v.at[c]], sem)
        b = pltpu.async_copy(row_v, xs_hbm.at[idx_v.at[2 + c]], sem)
        a.wait()
        b.wait()


@functools.lru_cache(maxsize=None)
def _sc_kernels(t_tokens):
    mesh = plsc.VectorSubcoreMesh(core_axis_name="c", subcore_axis_name="s")
    dispatch = pl.kernel(
        _sc_dispatch_body,
        mesh=mesh,
        out_type=jax.ShapeDtypeStruct((TOP_K * t_tokens, H), jnp.float32),
        scratch_types=[
            pltpu.VMEM((4, CH), jnp.int32),
            pltpu.VMEM((CH, H), jnp.float32),
            pltpu.SemaphoreType.DMA,
        ],
    )
    combine = pl.kernel(
        _sc_combine_body,
        mesh=mesh,
        out_type=jax.ShapeDtypeStruct((t_tokens, H), jnp.float32),
        scratch_types=[
            pltpu.VMEM((4, CH), jnp.int32),
            pltpu.VMEM((CH, 16), jnp.float32),
            pltpu.VMEM((CH, 16), jnp.float32),
            pltpu.VMEM((CH, H), jnp.float32),
            pltpu.VMEM((CH, H), jnp.float32),
            pltpu.SemaphoreType.DMA,
        ],
    )
    return dispatch, combine


def _sc_combine_body(ys_hbm, p0_hbm, p1_hbm, grw0_hbm, grw1_hbm, out_hbm,
                     idx_v, g0_v, g1_v, buf0, buf1, sem):
    base = _sc_worker_id() * TPW
    pltpu.sync_copy(p0_hbm.at[pl.ds(base, CH)], idx_v.at[0])
    pltpu.sync_copy(p0_hbm.at[pl.ds(base + CH, CH)], idx_v.at[1])
    pltpu.sync_copy(p1_hbm.at[pl.ds(base, CH)], idx_v.at[2])
    pltpu.sync_copy(p1_hbm.at[pl.ds(base + CH, CH)], idx_v.at[3])
    for c in range(TPW // CH):
        a = pltpu.async_copy(ys_hbm.at[idx_v.at[c]], buf0, sem)
        b = pltpu.async_copy(ys_hbm.at[idx_v.at[2 + c]], buf1, sem)
        pltpu.sync_copy(grw0_hbm.at[pl.ds(base + c * CH, CH)], g0_v)
        pltpu.sync_copy(grw1_hbm.at[pl.ds(base + c * CH, CH)], g1_v)
        a.wait()
        b.wait()

        def tok_body(j, carry):
            g0 = g0_v[j, :]
            g1 = g1_v[j, :]
            for k in range(H // 16):
                sl = pl.ds(k * 16, 16)
                buf0[j, sl] = buf0[j, sl] * g0 + buf1[j, sl] * g1
            return carry

        lax.fori_loop(0, CH, tok_body, 0)
        pltpu.sync_copy(buf0, out_hbm.at[pl.ds(base + c * CH, CH)])


def kernel(x, gate_w, c_fc_w, c_fc_b, c_proj_w, c_proj_b):
    orig_shape = x.shape
    xf = x.reshape(-1, H)
    t_tokens = xf.shape[0]
    n = t_tokens * TOP_K
    nt = n // TM

    router_logits = xf @ gate_w.T
    eids = jnp.arange(E, dtype=jnp.int32)
    v0 = jnp.max(router_logits, axis=-1)
    sel0 = jnp.argmax(router_logits, axis=-1).astype(jnp.int32)
    masked = jnp.where(eids[None, :] == sel0[:, None], -jnp.inf, router_logits)
    v1 = jnp.max(masked, axis=-1)
    sel1 = jnp.argmax(masked, axis=-1).astype(jnp.int32)
    rw0 = 1.0 / (1.0 + jnp.exp(v1 - v0))
    rw1 = 1.0 - rw0

    flat_sel = jnp.stack([sel0, sel1], axis=1).reshape(-1)
    sorted_idx = jnp.argsort(flat_sel)
    inv = jnp.zeros((n,), jnp.int32).at[sorted_idx].set(
        jnp.arange(n, dtype=jnp.int32))
    pos01 = inv.reshape(t_tokens, TOP_K)
    pos0, pos1 = pos01[:, 0], pos01[:, 1]
    counts = jnp.sum(flat_sel[:, None] == eids[None, :], axis=0)
    ends = jnp.cumsum(counts)
    starts = ends - counts

    t0 = starts // TM
    t1 = jnp.maximum(ends - 1, 0) // TM
    u = jnp.where(counts > 0, t1 - t0 + 1, 0)
    cum_u = jnp.cumsum(u)
    grid_steps = nt + E - 1
    s_idx = jnp.minimum(jnp.arange(grid_steps), cum_u[-1] - 1)
    eid = jnp.searchsorted(cum_u, s_idx, side="right").astype(jnp.int32)
    u_excl = cum_u - u
    step_tile = (t0[eid] + (s_idx - u_excl[eid])).astype(jnp.int32)
    is_pad = jnp.arange(grid_steps) >= cum_u[-1]
    step_st = jnp.where(is_pad, 0, starts[eid]).astype(jnp.int32)
    step_en = jnp.where(is_pad, 0, ends[eid]).astype(jnp.int32)

    sc_dispatch, sc_combine = _sc_kernels(t_tokens)
    xs = sc_dispatch(xf, pos0, pos1)
    ys = _grouped_mlp(xs, c_fc_w, c_fc_b, c_proj_w, c_proj_b,
                      step_tile, eid, step_st, step_en, grid_steps)
    grw0 = jnp.broadcast_to(rw0[:, None], (t_tokens, 16))
    grw1 = jnp.broadcast_to(rw1[:, None], (t_tokens, 16))
    out = sc_combine(ys, pos0, pos1, grw0, grw1)

    return (out.reshape(orig_shape), router_logits)
